# Initial kernel scaffold; baseline (speedup 1.0000x reference)
import jax
import jax.numpy as jnp
from jax.experimental import pallas as pl


def kernel(edge_index, edge_type, node_emb, rel_emb):
    raise NotImplementedError("write your pallas kernel here")


# speedup vs baseline: 2.0530x; 2.0530x over previous
import functools

import jax
import jax.numpy as jnp
from jax import lax
from jax.experimental import pallas as pl
from jax.experimental.pallas import tpu as pltpu
from jax.experimental.pallas import tpu_sc as plsc

_C = 128
_NW = 32
_LANES = 16


def _normalize_body(x_ref, o_ref):
    x = x_ref[...]
    s = jnp.sum(x * x, axis=1, keepdims=True)
    o_ref[...] = x / (jnp.sqrt(s) + 1e-12)


@functools.lru_cache(maxsize=None)
def _make_score_kernel(e_pad: int, n_rel: int, d: int):
    per_tile = e_pad // _NW
    n_chunks = per_tile // _C
    mesh = plsc.VectorSubcoreMesh(core_axis_name="c", subcore_axis_name="s")

    @functools.partial(
        pl.kernel,
        out_type=jax.ShapeDtypeStruct((e_pad,), jnp.float32),
        mesh=mesh,
        compiler_params=pltpu.CompilerParams(needs_layout_passes=False),
        scratch_types=[
            pltpu.VMEM((n_rel * d,), jnp.float32),
            pltpu.VMEM((_C,), jnp.int32),
            pltpu.VMEM((_C,), jnp.int32),
            pltpu.VMEM((_C, d), jnp.float32),
            pltpu.VMEM((_C, d), jnp.float32),
            pltpu.VMEM((_C,), jnp.int32),
            pltpu.VMEM((_C,), jnp.float32),
            pltpu.SemaphoreType.DMA,
            pltpu.SemaphoreType.DMA,
        ],
    )
    def score_k(emb_hbm, hidx_hbm, tidx_hbm, etype_hbm, rel_hbm, out_hbm,
                rel_v, hidx_v, tidx_v, hrows_v, trows_v, etype_v,
                scores_v, sem_h, sem_t):
        lane = lax.iota(jnp.int32, _LANES)
        wid = lax.axis_index("s") * 2 + lax.axis_index("c")
        base0 = wid * per_tile
        pltpu.sync_copy(rel_hbm, rel_v)

        @pl.loop(0, n_chunks)
        def _chunk(g):
            base = base0 + g * _C
            pltpu.sync_copy(hidx_hbm.at[pl.ds(base, _C)], hidx_v)
            pltpu.sync_copy(tidx_hbm.at[pl.ds(base, _C)], tidx_v)
            pltpu.sync_copy(etype_hbm.at[pl.ds(base, _C)], etype_v)
            cp_h = pltpu.async_copy(emb_hbm.at[hidx_v], hrows_v, sem_h)
            cp_t = pltpu.async_copy(emb_hbm.at[tidx_v], trows_v, sem_t)
            cp_h.wait()
            cp_t.wait()

            @pl.loop(0, _C // _LANES)
            def _grp(gg):
                res = jnp.zeros((_LANES,), jnp.float32)
                te_all = etype_v[pl.ds(gg * _LANES, _LANES)]
                for j in range(_LANES):
                    e = gg * _LANES + j
                    te_b = te_all.at[jnp.full((_LANES,), j, jnp.int32)].get(
                        mode="promise_in_bounds")
                    rbase = te_b * d + lane
                    acc = jnp.zeros((_LANES,), jnp.float32)
                    for k in range(d // _LANES):
                        sl = pl.ds(k * _LANES, _LANES)
                        acc = acc + (hrows_v[e, sl] * trows_v[e, sl]
                                     * plsc.load_gather(
                                         rel_v, [rbase + k * _LANES]))
                    for sh in (8, 4, 2, 1):
                        acc = acc + acc.at[lane ^ sh].get(
                            mode="promise_in_bounds")
                    res = jnp.where(lane == j, acc, res)
                scores_v[pl.ds(gg * _LANES, _LANES)] = res

            pltpu.sync_copy(scores_v, out_hbm.at[pl.ds(base, _C)])

    return score_k


def kernel(edge_index, edge_type, node_emb, rel_emb):
    n_nodes, d = node_emb.shape
    n_rel = rel_emb.shape[0]
    e = edge_type.shape[0]

    rows_per_block = 2000
    emb = pl.pallas_call(
        _normalize_body,
        grid=(n_nodes // rows_per_block,),
        in_specs=[pl.BlockSpec((rows_per_block, d), lambda i: (i, 0))],
        out_specs=pl.BlockSpec((rows_per_block, d), lambda i: (i, 0)),
        out_shape=jax.ShapeDtypeStruct((n_nodes, d), jnp.float32),
    )(node_emb)

    quantum = _NW * _C
    e_pad = ((e + quantum - 1) // quantum) * quantum
    pad = e_pad - e
    hidx = jnp.pad(edge_index[0].astype(jnp.int32), (0, pad))
    tidx = jnp.pad(edge_index[1].astype(jnp.int32), (0, pad))
    etype = jnp.pad(edge_type.astype(jnp.int32), (0, pad))

    scores = _make_score_kernel(e_pad, n_rel, d)(
        emb, hidx, tidx, etype, rel_emb.reshape(-1))
    return scores[:e]


# speedup vs baseline: 2.7648x; 1.3467x over previous
import functools

import jax
import jax.numpy as jnp
from jax import lax
from jax.experimental import pallas as pl
from jax.experimental.pallas import tpu as pltpu
from jax.experimental.pallas import tpu_sc as plsc

_C = 128
_NW = 32
_LANES = 16


def _normalize_body(x_ref, o_ref):
    x = x_ref[...]
    s = jnp.sum(x * x, axis=1, keepdims=True)
    o_ref[...] = x / (jnp.sqrt(s) + 1e-12)


@functools.lru_cache(maxsize=None)
def _make_score_kernel(e_pad: int, n_rel: int, d: int):
    per_tile = e_pad // _NW
    n_chunks = per_tile // _C
    assert n_chunks % 2 == 0 and n_chunks >= 4
    mesh = plsc.VectorSubcoreMesh(core_axis_name="c", subcore_axis_name="s")

    @functools.partial(
        pl.kernel,
        out_type=jax.ShapeDtypeStruct((e_pad,), jnp.float32),
        mesh=mesh,
        compiler_params=pltpu.CompilerParams(needs_layout_passes=False),
        scratch_types=[
            pltpu.VMEM((n_rel * d,), jnp.float32),
            pltpu.VMEM((3, _C), jnp.int32),
            pltpu.VMEM((3, _C), jnp.int32),
            pltpu.VMEM((_C, d), jnp.float32),
            pltpu.VMEM((_C, d), jnp.float32),
            pltpu.VMEM((_C, d), jnp.float32),
            pltpu.VMEM((_C, d), jnp.float32),
            pltpu.VMEM((_C,), jnp.float32),
            pltpu.VMEM((_C,), jnp.float32),
            pltpu.SemaphoreType.DMA,
            pltpu.SemaphoreType.DMA,
            pltpu.SemaphoreType.DMA,
            pltpu.SemaphoreType.DMA,
            pltpu.SemaphoreType.DMA,
            pltpu.SemaphoreType.DMA,
            pltpu.SemaphoreType.DMA,
            pltpu.SemaphoreType.DMA,
        ],
    )
    def score_k(emb_hbm, idx_hbm, rel_hbm, out_hbm,
                rel_v, idx0, idx1, hr0, hr1, tr0, tr1, sc0, sc1,
                si0, si1, sh0, sh1, st0, st1, so0, so1):
        idx_v = (idx0, idx1)
        hrows = (hr0, hr1)
        trows = (tr0, tr1)
        scores = (sc0, sc1)
        sem_i = (si0, si1)
        sem_h = (sh0, sh1)
        sem_t = (st0, st1)
        sem_o = (so0, so1)
        lane = lax.iota(jnp.int32, _LANES)

        wid = lax.axis_index("s") * 2 + lax.axis_index("c")
        base0 = wid * per_tile
        cid0 = wid * n_chunks
        pltpu.sync_copy(rel_hbm, rel_v)

        def issue_gathers(g, bi):
            pltpu.async_copy(emb_hbm.at[idx_v[bi].at[0]], hrows[bi],
                             sem_h[bi])
            pltpu.async_copy(emb_hbm.at[idx_v[bi].at[1]], trows[bi],
                             sem_t[bi])

        def wait_gathers(bi):
            pltpu.make_async_copy(emb_hbm.at[idx_v[bi].at[0]], hrows[bi],
                                  sem_h[bi]).wait()
            pltpu.make_async_copy(emb_hbm.at[idx_v[bi].at[1]], trows[bi],
                                  sem_t[bi]).wait()

        def compute(g, bi):
            @pl.loop(0, _C // _LANES)
            def _grp(gg):
                res = jnp.zeros((_LANES,), jnp.float32)
                te_all = idx_v[bi][2, pl.ds(gg * _LANES, _LANES)]
                for j in range(_LANES):
                    e = gg * _LANES + j
                    te_b = te_all.at[jnp.full((_LANES,), j, jnp.int32)].get(
                        mode="promise_in_bounds")
                    rbase = te_b * d + lane
                    acc = jnp.zeros((_LANES,), jnp.float32)
                    for k in range(d // _LANES):
                        sl = pl.ds(k * _LANES, _LANES)
                        acc = acc + (hrows[bi][e, sl] * trows[bi][e, sl]
                                     * plsc.load_gather(
                                         rel_v, [rbase + k * _LANES]))
                    for sh in (8, 4, 2, 1):
                        acc = acc + acc.at[lane ^ sh].get(
                            mode="promise_in_bounds")
                    res = jnp.where(lane == j, acc, res)
                scores[bi][pl.ds(gg * _LANES, _LANES)] = res

        pltpu.sync_copy(idx_hbm.at[cid0], idx_v[0])
        issue_gathers(0, 0)
        pltpu.async_copy(idx_hbm.at[cid0 + 1], idx_v[1], sem_i[1])

        @pl.loop(0, n_chunks, step=2)
        def _pair(g0):
            for b in range(2):
                g = g0 + b
                nb = 1 - b

                @pl.when(g + 1 < n_chunks)
                def _():
                    pltpu.make_async_copy(idx_hbm.at[cid0], idx_v[nb],
                                          sem_i[nb]).wait()
                    issue_gathers(g + 1, nb)

                wait_gathers(b)

                @pl.when(g >= 2)
                def _():
                    pltpu.make_async_copy(scores[b], out_hbm.at[pl.ds(0, _C)],
                                          sem_o[b]).wait()

                compute(g, b)

                @pl.when(g + 2 < n_chunks)
                def _():
                    pltpu.async_copy(idx_hbm.at[cid0 + g + 2], idx_v[b],
                                     sem_i[b])
                pltpu.async_copy(scores[b],
                                 out_hbm.at[pl.ds(base0 + g * _C, _C)],
                                 sem_o[b])

        for b in range(2):
            pltpu.make_async_copy(scores[b], out_hbm.at[pl.ds(0, _C)],
                                  sem_o[b]).wait()

    return score_k


def kernel(edge_index, edge_type, node_emb, rel_emb):
    n_nodes, d = node_emb.shape
    n_rel = rel_emb.shape[0]
    e = edge_type.shape[0]

    rows_per_block = 2000
    emb = pl.pallas_call(
        _normalize_body,
        grid=(n_nodes // rows_per_block,),
        in_specs=[pl.BlockSpec((rows_per_block, d), lambda i: (i, 0))],
        out_specs=pl.BlockSpec((rows_per_block, d), lambda i: (i, 0)),
        out_shape=jax.ShapeDtypeStruct((n_nodes, d), jnp.float32),
    )(node_emb)

    quantum = _NW * _C * 2
    e_pad = ((e + quantum - 1) // quantum) * quantum
    pad = e_pad - e
    hidx = jnp.pad(edge_index[0].astype(jnp.int32), (0, pad))
    tidx = jnp.pad(edge_index[1].astype(jnp.int32), (0, pad))
    etype = jnp.pad(edge_type.astype(jnp.int32), (0, pad))
    n_chunks_total = e_pad // _C
    packed = jnp.stack(
        [hidx.reshape(n_chunks_total, _C),
         tidx.reshape(n_chunks_total, _C),
         etype.reshape(n_chunks_total, _C)], axis=1)

    scores = _make_score_kernel(e_pad, n_rel, d)(
        emb, packed, rel_emb.reshape(-1))
    return scores[:e]
